# Initial kernel scaffold; baseline (speedup 1.0000x reference)
import jax
import jax.numpy as jnp
from jax.experimental import pallas as pl


def kernel(user_embed, item_embed, W, edge_index, edge_weight):
    raise NotImplementedError("write your pallas kernel here")


# speedup vs baseline: 4.2478x; 4.2478x over previous
import functools

import jax
import jax.numpy as jnp
from jax import lax
from jax.experimental import pallas as pl
from jax.experimental.pallas import tpu as pltpu
from jax.experimental.pallas import tpu_sc as plsc

N = 50000
HALF = 25000
D = 64
E_TOTAL = 800000
NS = 16
L = 16

EPT = E_TOTAL // NS
K = 128
NBUF = 3
NBATCH = 393
TRI = NBATCH // NBUF
PK = 3 * K
RPT = 1568
ACC_ROWS = NS * RPT
DUMMY = HALF
LAST = HALF - (NS - 1) * RPT

_mesh = plsc.VectorSubcoreMesh(core_axis_name="c", subcore_axis_name="s")


@functools.partial(
    pl.kernel,
    mesh=_mesh,
    compiler_params=pltpu.CompilerParams(use_tc_tiling_on_sc=False),
    out_type=jax.ShapeDtypeStruct((N, D), jnp.float32),
    scratch_types=[
        pltpu.VMEM((NBUF, PK), jnp.int32),
        pltpu.VMEM((K, D), jnp.float32),
        pltpu.VMEM((K, D), jnp.float32),
        pltpu.VMEM((K, D), jnp.float32),
        pltpu.VMEM((K,), jnp.int32),
        pltpu.VMEM((K,), jnp.int32),
        pltpu.VMEM((K,), jnp.int32),
        pltpu.VMEM_SHARED((ACC_ROWS, D), jnp.float32),
        pltpu.SemaphoreType.DMA((NBUF,)),
        pltpu.SemaphoreType.DMA((NBUF,)),
        pltpu.SemaphoreType.DMA((NBUF,)),
    ],
)
def _propagate(table, packed, zeros, out, idx_r, rows0, rows1, rows2,
               dstl0, dstl1, dstl2, acc, isem, gsem, ssem):
    rows_b = (rows0, rows1, rows2)
    dstl_b = (dstl0, dstl1, dstl2)
    c = lax.axis_index("c")
    s = lax.axis_index("s")

    pltpu.sync_copy(zeros, acc.at[pl.ds(s * RPT, RPT)])
    plsc.subcore_barrier()

    sc_off = c * HALF

    def idesc(b, j):
        return pltpu.make_async_copy(
            packed.at[pl.ds((s * NBATCH + b) * PK, PK)], idx_r.at[j],
            isem.at[j])

    def gdesc(j):
        return pltpu.make_async_copy(
            table.at[idx_r.at[j, pl.ds(0, K)]], rows_b[j], gsem.at[j])

    def sdesc(j):
        return pltpu.make_async_copy(rows_b[j], acc.at[dstl_b[j]], ssem.at[j])

    def process(j):
        for g in range(K // L):
            dj = idx_r[j, pl.ds(K + g * L, L)]
            lo = dj - sc_off
            ok = (lo >= 0) & (lo < HALF)
            dstl_b[j][pl.ds(g * L, L)] = jnp.where(ok, lo, DUMMY)
        for g in range(K // L):
            wg = lax.bitcast_convert_type(
                idx_r[j, pl.ds(2 * K + g * L, L)], jnp.float32)
            for j16 in range(L):
                e = g * L + j16
                wv = jnp.full((L,), wg[j16], jnp.float32)
                for q in range(D // L):
                    rows_b[j][e, pl.ds(q * L, L)] = (
                        rows_b[j][e, pl.ds(q * L, L)] * wv)

    idesc(0, 0).start()
    idesc(1, 1).start()
    idesc(2, 2).start()
    idesc(0, 0).wait()
    gdesc(0).start()
    idesc(1, 1).wait()
    gdesc(1).start()

    def tri(p, carry):
        for j in range(NBUF):
            b = p * NBUF + j
            kn = (j + 2) % NBUF
            gdesc(j).wait()
            process(j)
            sdesc(j).start(add=True)

            @pl.when(p < TRI - 1)
            def _():
                idesc(b + 3, j).start()

            if j == 0:
                @pl.when(p > 0)
                def _():
                    sdesc(kn).wait()
                idesc(b + 2, kn).wait()
                gdesc(kn).start()
            else:
                sdesc(kn).wait()

                @pl.when(p < TRI - 1)
                def _():
                    idesc(b + 2, kn).wait()
                    gdesc(kn).start()
        return carry

    lax.fori_loop(0, TRI, tri, None)
    sdesc((NBATCH - 1) % NBUF).wait()

    plsc.subcore_barrier()

    @pl.when(s < NS - 1)
    def _copy_full():
        pltpu.sync_copy(acc.at[pl.ds(s * RPT, RPT)],
                        out.at[pl.ds(sc_off + s * RPT, RPT)])

    @pl.when(s == NS - 1)
    def _copy_tail():
        pltpu.sync_copy(acc.at[pl.ds((NS - 1) * RPT, LAST)],
                        out.at[pl.ds(sc_off + (NS - 1) * RPT, LAST)])


BLK = 1000


def _finalize_body(e1_ref, e2_ref, w_ref, con_ref, bin_ref):
    e1 = e1_ref[...]
    e2 = e2_ref[...]
    con_ref[...] = jnp.concatenate([e1, e2], axis=1)
    wm = w_ref[...]
    p1 = jnp.dot(e1, wm, preferred_element_type=jnp.float32)
    p2 = jnp.dot(e2, wm, preferred_element_type=jnp.float32)
    bin_ref[...] = jnp.concatenate([jnp.sign(p1), jnp.sign(p2)], axis=1)


def _finalize(e_l1, e_l2, w, off):
    return pl.pallas_call(
        _finalize_body,
        grid=(HALF // BLK,),
        in_specs=[
            pl.BlockSpec((BLK, D), lambda i, o=off: (i + o, 0)),
            pl.BlockSpec((BLK, D), lambda i, o=off: (i + o, 0)),
            pl.BlockSpec((D, 32), lambda i: (0, 0)),
        ],
        out_specs=[
            pl.BlockSpec((BLK, 2 * D), lambda i: (i, 0)),
            pl.BlockSpec((BLK, D), lambda i: (i, 0)),
        ],
        out_shape=[
            jax.ShapeDtypeStruct((HALF, 2 * D), jnp.float32),
            jax.ShapeDtypeStruct((HALF, D), jnp.float32),
        ],
    )(e_l1, e_l2, w)


def kernel(user_embed, item_embed, W, edge_index, edge_weight):
    e0 = jnp.concatenate([user_embed, item_embed], axis=0)
    src = edge_index[0].astype(jnp.int32)
    dst = edge_index[1].astype(jnp.int32)
    wbits = jax.lax.bitcast_convert_type(
        edge_weight.astype(jnp.float32), jnp.int32)

    pad = NBATCH * K - EPT
    parts = []
    for arr in (src, dst, wbits):
        a = jnp.pad(arr.reshape(NS, EPT), ((0, 0), (0, pad)))
        parts.append(a.reshape(NS, NBATCH, K))
    packed = jnp.stack(parts, axis=2).reshape(-1)

    zeros = jnp.zeros((RPT, D), jnp.float32)

    e_l1 = _propagate(e0, packed, zeros)
    e_l2 = _propagate(e_l1, packed, zeros)

    con_users, bin_users = _finalize(e_l1, e_l2, W, 0)
    con_items, bin_items = _finalize(e_l1, e_l2, W, HALF // BLK)
    return (con_users, con_items, bin_users, bin_items)
